# Initial kernel scaffold; baseline (speedup 1.0000x reference)
import jax
import jax.numpy as jnp
from jax.experimental import pallas as pl


def kernel(edge_index, edge_weight, nodes_yield_rate, nodes_traffic, nodes_cost):
    raise NotImplementedError("write your pallas kernel here")


# speedup vs baseline: 174.9403x; 174.9403x over previous
import dataclasses
import functools

import jax
import jax.numpy as jnp
from jax import lax
from jax.experimental import pallas as pl
from jax.experimental.pallas import tpu as pltpu
from jax.experimental.pallas import tpu_sc as plsc

N_NODES = 100000
N_EDGES = 6400000
PEN = 0.01

NC, NS, L = 2, 16, 16
NW = NC * NS
NPAD = 100352
SLICE = NPAD // NS
EPAD = 6553600
ROWS = EPAD // 128
ROWS_PER_TILE = ROWS // NW
CHUNK_ROWS = 16
NCHUNKS = ROWS_PER_TILE // CHUNK_ROWS


def _sc_edge_kernel(src_hbm, dst_hbm, w_hbm, traffic_hbm, out_hbm,
                    table, srcb, dstb, wb, tneg, tpos, zbuf, acc):
    c = lax.axis_index("c")
    s = lax.axis_index("s")
    wid = c * NS + s

    pltpu.sync_copy(traffic_hbm, table)

    @pl.loop(0, SLICE, step=L)
    def _(i):
        zbuf[pl.ds(i, L)] = jnp.zeros((L,), jnp.float32)

    pltpu.sync_copy(zbuf, acc.at[pl.ds(s * SLICE, SLICE)])
    plsc.subcore_barrier()

    base_row = wid * ROWS_PER_TILE

    @pl.loop(0, NCHUNKS)
    def _(k):
        r0 = base_row + k * CHUNK_ROWS
        pltpu.sync_copy(src_hbm.at[pl.ds(r0, CHUNK_ROWS)], srcb)
        pltpu.sync_copy(dst_hbm.at[pl.ds(r0, CHUNK_ROWS)], dstb)
        pltpu.sync_copy(w_hbm.at[pl.ds(r0, CHUNK_ROWS)], wb)

        @pl.loop(0, CHUNK_ROWS)
        def _(j):
            for cc in range(128 // L):
                sl = pl.ds(cc * L, L)
                si = srcb[j, sl]
                di = dstb[j, sl]
                wv = wb[j, sl]
                sv = plsc.load_gather(table, [si])
                dv = plsc.load_gather(table, [di])
                t = jnp.abs(sv - dv) * (wv * PEN)
                tpos[j, sl] = t
                tneg[j, sl] = -t
            pltpu.sync_copy(tneg.at[j], acc.at[srcb.at[j]], add=True)
            pltpu.sync_copy(tpos.at[j], acc.at[dstb.at[j]], add=True)

    plsc.subcore_barrier()
    pltpu.sync_copy(acc.at[pl.ds(s * SLICE, SLICE)], zbuf)
    pltpu.sync_copy(zbuf, out_hbm.at[pl.ds(c * NPAD + s * SLICE, SLICE)])


def _tc_combine_kernel(acc_ref, t_ref, y_ref, c_ref, new_ref, eff_ref):
    new = t_ref[...] + acc_ref[0] + acc_ref[1]
    new_ref[...] = new
    eff = jnp.sum(y_ref[...] * new) - jnp.sum(c_ref[...])
    eff_ref[...] = eff.reshape(1, 1)


def kernel(edge_index, edge_weight, nodes_yield_rate, nodes_traffic, nodes_cost):
    src = edge_index[0].astype(jnp.int32)
    dst = edge_index[1].astype(jnp.int32)
    w = edge_weight.astype(jnp.float32)
    pad = EPAD - N_EDGES
    src2d = jnp.pad(src, (0, pad)).reshape(ROWS, 128)
    dst2d = jnp.pad(dst, (0, pad)).reshape(ROWS, 128)
    w2d = jnp.pad(w, (0, pad)).reshape(ROWS, 128)

    mesh = plsc.VectorSubcoreMesh(core_axis_name="c", subcore_axis_name="s")
    cp = pltpu.CompilerParams()
    if "needs_layout_passes" in pltpu.CompilerParams.__dataclass_fields__:
        cp = dataclasses.replace(cp, needs_layout_passes=False)
    sc_call = functools.partial(
        pl.kernel,
        compiler_params=cp,
        out_type=jax.ShapeDtypeStruct((NC * NPAD,), jnp.float32),
        mesh=mesh,
        scratch_types=[
            pltpu.VMEM((N_NODES,), jnp.float32),
            pltpu.VMEM((CHUNK_ROWS, 128), jnp.int32),
            pltpu.VMEM((CHUNK_ROWS, 128), jnp.int32),
            pltpu.VMEM((CHUNK_ROWS, 128), jnp.float32),
            pltpu.VMEM((CHUNK_ROWS, 128), jnp.float32),
            pltpu.VMEM((CHUNK_ROWS, 128), jnp.float32),
            pltpu.VMEM((SLICE,), jnp.float32),
            pltpu.VMEM_SHARED((NPAD,), jnp.float32),
        ],
    )(_sc_edge_kernel)
    accs = sc_call(src2d, dst2d, w2d, nodes_traffic)

    npad = NPAD - N_NODES
    t2 = jnp.pad(nodes_traffic, (0, npad)).reshape(NPAD // 128, 128)
    y2 = jnp.pad(nodes_yield_rate, (0, npad)).reshape(NPAD // 128, 128)
    c2 = jnp.pad(nodes_cost, (0, npad)).reshape(NPAD // 128, 128)
    acc3 = accs.reshape(NC, NPAD // 128, 128)

    new2, eff = pl.pallas_call(
        _tc_combine_kernel,
        out_shape=[
            jax.ShapeDtypeStruct((NPAD // 128, 128), jnp.float32),
            jax.ShapeDtypeStruct((1, 1), jnp.float32),
        ],
    )(acc3, t2, y2, c2)

    new_traffic = new2.reshape(NPAD)[:N_NODES]
    return (new_traffic, eff[0, 0])
